# Initial kernel scaffold; baseline (speedup 1.0000x reference)
import jax
import jax.numpy as jnp
from jax.experimental import pallas as pl


def kernel(x, edge_index, W1, b1, W2, b2):
    raise NotImplementedError("write your pallas kernel here")


# speedup vs baseline: 8.2206x; 8.2206x over previous
import functools

import jax
import jax.numpy as jnp
from jax import lax
from jax.experimental import pallas as pl
from jax.experimental.pallas import tpu as pltpu
from jax.experimental.pallas import tpu_sc as plsc

N = 10000
E = 160000
D = 256
DH = 128
NP = 10240
STRIPE = NP // 16
G = 128
CH = 79
TPT = CH * G
EPAD = 16 * TPT

_mesh = plsc.VectorSubcoreMesh(core_axis_name="c", subcore_axis_name="s")


KDEG = 6400
AHI = NP // 128


def _deg_body(col_ref, o_ref):
    i = pl.program_id(0)
    c = col_ref[...]
    hi = jax.lax.shift_right_logical(c, 7)
    lo = jnp.bitwise_and(c, 127)
    oh_hi = (hi == jax.lax.broadcasted_iota(jnp.int32, (1, AHI), 1)
             ).astype(jnp.bfloat16)
    oh_lo = (lo == jax.lax.broadcasted_iota(jnp.int32, (1, 128), 1)
             ).astype(jnp.bfloat16)
    part = lax.dot_general(oh_hi, oh_lo, (((0,), (0,)), ((), ())),
                           preferred_element_type=jnp.float32)

    @pl.when(i == 0)
    def _():
        o_ref[...] = jnp.zeros_like(o_ref)

    o_ref[...] += part


def _tc_degree(col2):
    return pl.pallas_call(
        _deg_body,
        grid=(E // KDEG,),
        in_specs=[pl.BlockSpec((KDEG, 1), lambda i: (i, 0))],
        out_specs=pl.BlockSpec((AHI, 128), lambda i: (0, 0)),
        out_shape=jax.ShapeDtypeStruct((AHI, 128), jnp.float32),
    )(col2)


@functools.partial(
    pl.kernel,
    mesh=_mesh,
    out_type=[jax.ShapeDtypeStruct((NP, DH), jnp.float32)] * 2,
    scratch_types=[
        pltpu.VMEM((CH, G), jnp.int32),
        pltpu.VMEM((CH, G), jnp.int32),
        pltpu.VMEM((G, DH), jnp.float32),
        pltpu.VMEM_SHARED((NP, DH), jnp.float32),
    ],
)
def _sc_edge_agg(hsa_hbm, hsb_hbm, rowp_hbm, colp_hbm, zc_hbm,
                 oa_hbm, ob_hbm, rowv, colv, buf, acc):
    c = lax.axis_index("c")
    s = lax.axis_index("s")

    pltpu.sync_copy(zc_hbm, acc.at[pl.ds(s * STRIPE, STRIPE)])
    pltpu.sync_copy(rowp_hbm.at[s], rowv)
    pltpu.sync_copy(colp_hbm.at[s], colv)
    plsc.subcore_barrier()

    for cv, hsr in ((0, hsa_hbm), (1, hsb_hbm)):
        @pl.when(c == cv)
        def _(hsr=hsr):
            def body(j, carry):
                pltpu.sync_copy(hsr.at[rowv.at[j]], buf)
                pltpu.sync_copy(buf, acc.at[colv.at[j]], add=True)
                return carry

            lax.fori_loop(0, CH, body, 0)

    plsc.subcore_barrier()
    for cv, outr in ((0, oa_hbm), (1, ob_hbm)):
        @pl.when(c == cv)
        def _(outr=outr):
            pltpu.sync_copy(acc.at[pl.ds(s * STRIPE, STRIPE)],
                            outr.at[pl.ds(s * STRIPE, STRIPE)])


BN = 512


def _mm_body(x_ref, w_ref, deg_ref, oa_ref, ob_ref):
    h = jnp.dot(x_ref[...], w_ref[...], preferred_element_type=jnp.float32)
    dinv = lax.rsqrt(deg_ref[...] + 1.0)
    hs = h * dinv
    oa_ref[...] = hs[:, :DH]
    ob_ref[...] = hs[:, DH:]


def _tc_matmul_scale(xp, w, degf):
    return pl.pallas_call(
        _mm_body,
        grid=(NP // BN,),
        in_specs=[
            pl.BlockSpec((BN, D), lambda i: (i, 0)),
            pl.BlockSpec((D, D), lambda i: (0, 0)),
            pl.BlockSpec((BN, 1), lambda i: (i, 0)),
        ],
        out_specs=[
            pl.BlockSpec((BN, DH), lambda i: (i, 0)),
            pl.BlockSpec((BN, DH), lambda i: (i, 0)),
        ],
        out_shape=[jax.ShapeDtypeStruct((NP, DH), jnp.float32)] * 2,
    )(xp, w, degf)


def _combine_body(relu, aa_ref, ab_ref, ha_ref, hb_ref, deg_ref, b_ref, o_ref):
    dinv = lax.rsqrt(deg_ref[...] + 1.0)
    ya = (aa_ref[...] + ha_ref[...]) * dinv + b_ref[0, :DH]
    yb = (ab_ref[...] + hb_ref[...]) * dinv + b_ref[0, DH:]
    y = jnp.concatenate([ya, yb], axis=1)
    if relu:
        y = jnp.maximum(y, 0.0)
    o_ref[...] = y


def _tc_combine(agg_a, agg_b, hs_a, hs_b, degf, b, relu):
    return pl.pallas_call(
        functools.partial(_combine_body, relu),
        grid=(NP // BN,),
        in_specs=[
            pl.BlockSpec((BN, DH), lambda i: (i, 0)),
            pl.BlockSpec((BN, DH), lambda i: (i, 0)),
            pl.BlockSpec((BN, DH), lambda i: (i, 0)),
            pl.BlockSpec((BN, DH), lambda i: (i, 0)),
            pl.BlockSpec((BN, 1), lambda i: (i, 0)),
            pl.BlockSpec((1, D), lambda i: (0, 0)),
        ],
        out_specs=pl.BlockSpec((BN, D), lambda i: (i, 0)),
        out_shape=jax.ShapeDtypeStruct((NP, D), jnp.float32),
    )(agg_a, agg_b, hs_a, hs_b, degf, b.reshape(1, D))


def kernel(x, edge_index, W1, b1, W2, b2):
    row0 = edge_index[0]
    col0 = edge_index[1]
    rowp = jnp.pad(row0, (0, EPAD - E)).reshape(16, CH, G)
    colp = jnp.pad(col0, (0, EPAD - E), constant_values=N).reshape(16, CH, G)
    xp = jnp.pad(x, ((0, NP - N), (0, 0)))
    zc = jnp.zeros((STRIPE, DH), jnp.float32)

    degf = _tc_degree(col0.reshape(E, 1)).reshape(NP, 1)

    hs1a, hs1b = _tc_matmul_scale(xp, W1, degf)
    agg1a, agg1b = _sc_edge_agg(hs1a, hs1b, rowp, colp, zc)
    h1 = _tc_combine(agg1a, agg1b, hs1a, hs1b, degf, b1, relu=True)

    hs2a, hs2b = _tc_matmul_scale(h1, W2, degf)
    agg2a, agg2b = _sc_edge_agg(hs2a, hs2b, rowp, colp, zc)
    out = _tc_combine(agg2a, agg2b, hs2a, hs2b, degf, b2, relu=False)

    return out[:N]
